# baseline (speedup 1.0000x reference)
import functools

import jax
import jax.numpy as jnp
from jax import lax
from jax.experimental import pallas as pl
from jax.experimental.pallas import tpu as pltpu
from jax.experimental.pallas import tpu_sc as plsc

N_NODES = 10000
N_EDGES = 320000
D_FEAT = 128

NC = 2
NS = 16
L = 16
NW = NC * NS
E_PER_W = N_EDGES // NW
C = 80
NG = C // L
N_CHUNK = E_PER_W // C


def _sqrt16(y):
    i = lax.bitcast_convert_type(y, jnp.int32)
    i = jnp.int32(0x1FBD1DF5) + lax.shift_right_logical(i, 1)
    g = lax.bitcast_convert_type(i, jnp.float32)
    g = 0.5 * (g + y / g)
    g = 0.5 * (g + y / g)
    g = 0.5 * (g + y / g)
    return g


def _score_kernel(x_hbm, src_hbm, dst_hbm, out_hbm,
                  sidx_v, didx_v, head_v, tail_v, out_v, sem_h, sem_t):
    wid = lax.axis_index("s") * NC + lax.axis_index("c")
    base = wid * E_PER_W

    row_idx = [lax.iota(jnp.int32, L) + g * L for g in range(NG)]

    def chunk_body(ci, carry):
        off = pl.multiple_of(base + ci * C, 8)
        pltpu.sync_copy(src_hbm.at[pl.ds(off, C)], sidx_v)
        pltpu.sync_copy(dst_hbm.at[pl.ds(off, C)], didx_v)
        cp_h = pltpu.async_copy(x_hbm.at[sidx_v], head_v, sem_h)
        cp_t = pltpu.async_copy(x_hbm.at[didx_v], tail_v, sem_t)
        cp_h.wait()
        cp_t.wait()

        def feat_body(d, accs):
            col = jnp.full((L,), d, jnp.int32)
            out = []
            for g in range(NG):
                h = plsc.load_gather(head_v, [row_idx[g], col])
                t = plsc.load_gather(tail_v, [row_idx[g], col])
                m = h * t
                out.append(accs[g] + m * m)
            return tuple(out)

        accs = lax.fori_loop(
            0, D_FEAT, feat_body,
            tuple(jnp.zeros((L,), jnp.float32) for _ in range(NG)))
        for g in range(NG):
            out_v[pl.ds(g * L, L)] = _sqrt16(accs[g])
        pltpu.sync_copy(out_v, out_hbm.at[pl.ds(off, C)])
        return carry

    lax.fori_loop(0, N_CHUNK, chunk_body, 0)


@jax.jit
def kernel(x, edge_index):
    src = edge_index[0]
    dst = edge_index[1]
    mesh = plsc.VectorSubcoreMesh(
        core_axis_name="c", subcore_axis_name="s", num_cores=NC, num_subcores=NS)
    f = functools.partial(
        pl.kernel,
        out_type=jax.ShapeDtypeStruct((N_EDGES,), jnp.float32),
        mesh=mesh,
        scratch_types=[
            pltpu.VMEM((C,), jnp.int32),
            pltpu.VMEM((C,), jnp.int32),
            pltpu.VMEM((C, D_FEAT), jnp.float32),
            pltpu.VMEM((C, D_FEAT), jnp.float32),
            pltpu.VMEM((C,), jnp.float32),
            pltpu.SemaphoreType.DMA,
            pltpu.SemaphoreType.DMA,
        ],
        compiler_params=pltpu.CompilerParams(needs_layout_passes=False),
    )(_score_kernel)
    return f(x, src, dst)


# speedup vs baseline: 1.2401x; 1.2401x over previous
import functools

import jax
import jax.numpy as jnp
from jax import lax
from jax.experimental import pallas as pl
from jax.experimental.pallas import tpu as pltpu
from jax.experimental.pallas import tpu_sc as plsc

N_NODES = 10000
N_EDGES = 320000
D_FEAT = 128

NC = 2
NS = 16
L = 16
NW = NC * NS
E_PER_W = N_EDGES // NW
C = 80
NG = C // L
N_CHUNK = E_PER_W // C
UNROLL = 4


def _sqrt16(y):
    i = lax.bitcast_convert_type(y, jnp.int32)
    i = jnp.int32(0x1FBD1DF5) + lax.shift_right_logical(i, 1)
    g = lax.bitcast_convert_type(i, jnp.float32)
    g = 0.5 * (g + y / g)
    g = 0.5 * (g + y / g)
    g = 0.5 * (g + y / g)
    return g


def _score_kernel(x_hbm, src_hbm, dst_hbm, out_hbm,
                  sidx_v, didx_v, out_v, head_v, tail_v, sems):
    wid = lax.axis_index("s") * NC + lax.axis_index("c")
    base = pl.multiple_of(wid * E_PER_W, 8)

    pltpu.sync_copy(src_hbm.at[pl.ds(base, E_PER_W)], sidx_v)
    pltpu.sync_copy(dst_hbm.at[pl.ds(base, E_PER_W)], didx_v)

    row_idx = [lax.iota(jnp.int32, L) + g * L for g in range(NG)]

    def start_gathers(ci, b):
        off = pl.multiple_of(ci * C, 8)
        pltpu.async_copy(x_hbm.at[sidx_v.at[pl.ds(off, C)]], head_v.at[b],
                         sems.at[b])
        pltpu.async_copy(x_hbm.at[didx_v.at[pl.ds(off, C)]], tail_v.at[b],
                         sems.at[b])

    def drain(b):
        pltpu.make_async_copy(x_hbm.at[pl.ds(0, C)], head_v.at[b],
                              sems.at[b]).wait()
        pltpu.make_async_copy(x_hbm.at[pl.ds(0, C)], tail_v.at[b],
                              sems.at[b]).wait()

    def compute(ci, b):
        hb = head_v.at[b]
        tb = tail_v.at[b]

        def feat_body(dj, accs):
            out = list(accs)
            for k in range(UNROLL):
                col = jnp.full((L,), dj * UNROLL + k, jnp.int32)
                for g in range(NG):
                    h = plsc.load_gather(hb, [row_idx[g], col])
                    t = plsc.load_gather(tb, [row_idx[g], col])
                    m = h * t
                    out[g] = out[g] + m * m
            return tuple(out)

        accs = lax.fori_loop(
            0, D_FEAT // UNROLL, feat_body,
            tuple(jnp.zeros((L,), jnp.float32) for _ in range(NG)))
        obase = ci * C
        for g in range(NG):
            out_v[pl.ds(obase + g * L, L)] = _sqrt16(accs[g])

    start_gathers(0, 0)

    def pair_body(j, carry):
        drain(0)
        start_gathers(2 * j + 1, 1)
        compute(2 * j, 0)
        drain(1)
        start_gathers(2 * j + 2, 0)
        compute(2 * j + 1, 1)
        return carry

    lax.fori_loop(0, (N_CHUNK - 1) // 2, pair_body, 0)
    drain(0)
    compute(N_CHUNK - 1, 0)

    pltpu.sync_copy(out_v, out_hbm.at[pl.ds(base, E_PER_W)])


@jax.jit
def kernel(x, edge_index):
    src = edge_index[0]
    dst = edge_index[1]
    mesh = plsc.VectorSubcoreMesh(
        core_axis_name="c", subcore_axis_name="s", num_cores=NC, num_subcores=NS)
    f = functools.partial(
        pl.kernel,
        out_type=jax.ShapeDtypeStruct((N_EDGES,), jnp.float32),
        mesh=mesh,
        scratch_types=[
            pltpu.VMEM((E_PER_W,), jnp.int32),
            pltpu.VMEM((E_PER_W,), jnp.int32),
            pltpu.VMEM((E_PER_W,), jnp.float32),
            pltpu.VMEM((2, C, D_FEAT), jnp.float32),
            pltpu.VMEM((2, C, D_FEAT), jnp.float32),
            pltpu.SemaphoreType.DMA((2,)),
        ],
        compiler_params=pltpu.CompilerParams(needs_layout_passes=False),
    )(_score_kernel)
    return f(x, src, dst)


# speedup vs baseline: 6.7053x; 5.4070x over previous
import functools

import jax
import jax.numpy as jnp
from jax import lax
from jax.experimental import pallas as pl
from jax.experimental.pallas import tpu as pltpu
from jax.experimental.pallas import tpu_sc as plsc

N_NODES = 10000
N_EDGES = 320000
D_FEAT = 128

NC = 2
NS = 16
L = 16
NW = NC * NS
E_PER_W = N_EDGES // NW
C = 80
N_CHUNK = E_PER_W // C
UNROLL = 4
NJ = D_FEAT // L
PAD_W = 17


def _sqrt16(y):
    i = lax.bitcast_convert_type(y, jnp.int32)
    i = jnp.int32(0x1FBD1DF5) + lax.shift_right_logical(i, 1)
    g = lax.bitcast_convert_type(i, jnp.float32)
    g = 0.5 * (g + y / g)
    g = 0.5 * (g + y / g)
    g = 0.5 * (g + y / g)
    return g


def _edge_acc(hb, tb, e):
    parts = []
    for j in range(NJ):
        h = hb[e, pl.ds(j * L, L)]
        t = tb[e, pl.ds(j * L, L)]
        m = h * t
        parts.append(m * m)
    while len(parts) > 1:
        parts = [a + b for a, b in zip(parts[::2], parts[1::2])]
    return parts[0]


def _score_kernel(x_hbm, src_hbm, dst_hbm, out_hbm,
                  sidx_v, didx_v, out_v, head_v, tail_v, pad_v, sems):
    wid = lax.axis_index("s") * NC + lax.axis_index("c")
    base = pl.multiple_of(wid * E_PER_W, 8)

    pltpu.sync_copy(src_hbm.at[pl.ds(base, E_PER_W)], sidx_v)
    pltpu.sync_copy(dst_hbm.at[pl.ds(base, E_PER_W)], didx_v)

    def start_gathers(ci, b):
        off = pl.multiple_of(ci * C, 8)
        pltpu.async_copy(x_hbm.at[sidx_v.at[pl.ds(off, C)]], head_v.at[b],
                         sems.at[b])
        pltpu.async_copy(x_hbm.at[didx_v.at[pl.ds(off, C)]], tail_v.at[b],
                         sems.at[b])

    def drain(b):
        pltpu.make_async_copy(x_hbm.at[pl.ds(0, C)], head_v.at[b],
                              sems.at[b]).wait()
        pltpu.make_async_copy(x_hbm.at[pl.ds(0, C)], tail_v.at[b],
                              sems.at[b]).wait()

    col_base = lax.iota(jnp.int32, L) * PAD_W

    def compute(ci, b):
        hb = head_v.at[b]
        tb = tail_v.at[b]
        obase = ci * C

        def group_body(g, carry):
            gbase = g * L

            def edge_body(eu, carry2):
                for k in range(UNROLL):
                    e = gbase + eu * UNROLL + k
                    pad_v[pl.ds((eu * UNROLL + k) * PAD_W, L)] = \
                        _edge_acc(hb, tb, e)
                return carry2

            lax.fori_loop(0, L // UNROLL, edge_body, 0)

            cols = [plsc.load_gather(pad_v, [col_base + j]) for j in range(L)]
            while len(cols) > 1:
                cols = [a + b for a, b in zip(cols[::2], cols[1::2])]
            out_v[pl.ds(obase + gbase, L)] = _sqrt16(cols[0])
            return carry

        lax.fori_loop(0, C // L, group_body, 0)

    start_gathers(0, 0)

    def pair_body(j, carry):
        drain(0)
        start_gathers(2 * j + 1, 1)
        compute(2 * j, 0)
        drain(1)
        start_gathers(2 * j + 2, 0)
        compute(2 * j + 1, 1)
        return carry

    lax.fori_loop(0, (N_CHUNK - 1) // 2, pair_body, 0)
    drain(0)
    compute(N_CHUNK - 1, 0)

    pltpu.sync_copy(out_v, out_hbm.at[pl.ds(base, E_PER_W)])


@jax.jit
def kernel(x, edge_index):
    src = edge_index[0]
    dst = edge_index[1]
    mesh = plsc.VectorSubcoreMesh(
        core_axis_name="c", subcore_axis_name="s", num_cores=NC, num_subcores=NS)
    f = functools.partial(
        pl.kernel,
        out_type=jax.ShapeDtypeStruct((N_EDGES,), jnp.float32),
        mesh=mesh,
        scratch_types=[
            pltpu.VMEM((E_PER_W,), jnp.int32),
            pltpu.VMEM((E_PER_W,), jnp.int32),
            pltpu.VMEM((E_PER_W,), jnp.float32),
            pltpu.VMEM((2, C, D_FEAT), jnp.float32),
            pltpu.VMEM((2, C, D_FEAT), jnp.float32),
            pltpu.VMEM((L * PAD_W,), jnp.float32),
            pltpu.SemaphoreType.DMA((2,)),
        ],
        compiler_params=pltpu.CompilerParams(needs_layout_passes=False),
    )(_score_kernel)
    return f(x, src, dst)
